# baseline (speedup 1.0000x reference)
import functools

import jax
import jax.numpy as jnp
from jax import lax
from jax.experimental import pallas as pl
from jax.experimental.pallas import tpu as pltpu
from jax.experimental.pallas import tpu_sc as plsc

B = 1
D = 128
N = 10000
K = 16

NW = 32
NPAD = 10240
NODES_PER_W = NPAD // NW
CHUNK = 8
NCHUNK = NODES_PER_W // CHUNK
IDX_PER_CHUNK = CHUNK * K
NB = 5120
NBLK = NPAD // NB
ROWS_PER_TILE = NPAD // 16


def _mm_body(x_ref, w_ref, yt_ref):
    xb = x_ref[...]
    w1 = w_ref[:, :D]
    dn = (((0,), (1,)), ((), ()))
    yt_ref[...] = lax.dot_general(xb, w1, dn, preferred_element_type=jnp.float32)


def _add_body(x_ref, m_ref, w_ref, b_ref, o_ref):
    xb = x_ref[...]
    wz = w_ref[:, D:] - w_ref[:, :D]
    z = lax.dot_general(wz, xb, (((1,), (0,)), ((), ())),
                        preferred_element_type=jnp.float32) + b_ref[...].T
    o_ref[...] = xb + jnp.maximum(z + m_ref[...].T, 0.0)


def _sc_knn(yt_hbm, idx_hbm, out_hbm, idx_v, gbuf_a, gbuf_b, obuf_a, obuf_b,
            yt_sp, sem_a, sem_b):
    cid = lax.axis_index("c")
    sid = lax.axis_index("s")
    wid = sid * 2 + cid
    base = wid * NODES_PER_W
    pltpu.sync_copy(yt_hbm.at[pl.ds(sid * ROWS_PER_TILE, ROWS_PER_TILE)],
                    yt_sp.at[pl.ds(sid * ROWS_PER_TILE, ROWS_PER_TILE)])
    pltpu.sync_copy(idx_hbm.at[wid], idx_v)
    plsc.subcore_barrier()

    def issue(jc, buf, sem):
        pltpu.async_copy(yt_sp.at[idx_v.at[jc]], buf, sem)

    def wait(buf, sem):
        pltpu.make_async_copy(yt_sp.at[pl.ds(0, IDX_PER_CHUNK)], buf, sem).wait()

    def compute_store(j, buf, obuf):
        def node_body(i, _):
            r0 = i * K
            for g in range(D // 16):
                sl = pl.ds(g * 16, 16)
                m = buf[r0, sl]
                for k in range(1, K):
                    m = jnp.maximum(m, buf[r0 + k, sl])
                obuf[i, sl] = m
            return 0

        lax.fori_loop(0, CHUNK, node_body, 0)
        pltpu.sync_copy(obuf, out_hbm.at[pl.ds(base + j * CHUNK, CHUNK)])

    issue(0, gbuf_a, sem_a)
    issue(1, gbuf_b, sem_b)

    def chunk_body(jj, _):
        j0 = jj * 2
        wait(gbuf_a, sem_a)
        compute_store(j0, gbuf_a, obuf_a)
        issue(jnp.minimum(j0 + 2, NCHUNK - 1), gbuf_a, sem_a)
        wait(gbuf_b, sem_b)
        compute_store(j0 + 1, gbuf_b, obuf_b)
        issue(jnp.minimum(j0 + 3, NCHUNK - 1), gbuf_b, sem_b)
        return 0

    lax.fori_loop(0, NCHUNK // 2, chunk_body, 0)
    wait(gbuf_a, sem_a)
    wait(gbuf_b, sem_b)


def kernel(x, idx, W, b):
    x2 = x[0]
    idx_flat = idx[0].astype(jnp.int32).reshape(-1)
    idx3 = jnp.pad(idx_flat, (0, (NPAD - N) * K)).reshape(NW, NCHUNK, IDX_PER_CHUNK)
    b2 = b.reshape(1, D)

    yt = pl.pallas_call(
        _mm_body,
        grid=(1,),
        in_specs=[
            pl.BlockSpec((D, N), lambda i: (0, 0)),
            pl.BlockSpec((D, 2 * D), lambda i: (0, 0)),
        ],
        out_specs=pl.BlockSpec((N, D), lambda i: (0, 0)),
        out_shape=jax.ShapeDtypeStruct((NPAD, D), jnp.float32),
    )(x2, W)

    mesh = plsc.VectorSubcoreMesh(core_axis_name="c", subcore_axis_name="s")
    mt = pl.kernel(
        _sc_knn,
        out_type=jax.ShapeDtypeStruct((NPAD, D), jnp.float32),
        mesh=mesh,
        scratch_types=[
            pltpu.VMEM((NCHUNK, IDX_PER_CHUNK), jnp.int32),
            pltpu.VMEM((IDX_PER_CHUNK, D), jnp.float32),
            pltpu.VMEM((IDX_PER_CHUNK, D), jnp.float32),
            pltpu.VMEM((CHUNK, D), jnp.float32),
            pltpu.VMEM((CHUNK, D), jnp.float32),
            pltpu.VMEM_SHARED((NPAD, D), jnp.float32),
            pltpu.SemaphoreType.DMA,
            pltpu.SemaphoreType.DMA,
        ],
    )(yt, idx3)

    out = pl.pallas_call(
        _add_body,
        grid=(1,),
        in_specs=[
            pl.BlockSpec((D, N), lambda i: (0, 0)),
            pl.BlockSpec((N, D), lambda i: (0, 0)),
            pl.BlockSpec((D, 2 * D), lambda i: (0, 0)),
            pl.BlockSpec((1, D), lambda i: (0, 0)),
        ],
        out_specs=pl.BlockSpec((D, N), lambda i: (0, 0)),
        out_shape=jax.ShapeDtypeStruct((D, N), jnp.float32),
    )(x2, mt, W, b2)

    return out[None]


# speedup vs baseline: 1.0271x; 1.0271x over previous
import functools

import jax
import jax.numpy as jnp
from jax import lax
from jax.experimental import pallas as pl
from jax.experimental.pallas import tpu as pltpu
from jax.experimental.pallas import tpu_sc as plsc

B = 1
D = 128
N = 10000
K = 16

NW = 32
NPAD = 10240
NODES_PER_W = NPAD // NW
CHUNK = 8
NCHUNK = NODES_PER_W // CHUNK
IDX_PER_CHUNK = CHUNK * K
NB = 5120
NBLK = NPAD // NB
ROWS_PER_TILE = NPAD // 16


def _mm_body(x_ref, w_ref, yt_ref):
    xb = x_ref[...]
    w1 = w_ref[:, :D]
    dn = (((0,), (1,)), ((), ()))
    yt_ref[...] = lax.dot_general(xb, w1, dn, preferred_element_type=jnp.float32)


def _add_body(x_ref, m_ref, w_ref, b_ref, o_ref):
    xb = x_ref[...]
    wz = w_ref[:, D:] - w_ref[:, :D]
    z = lax.dot_general(wz, xb, (((1,), (0,)), ((), ())),
                        preferred_element_type=jnp.float32) + b_ref[...].T
    o_ref[...] = xb + jnp.maximum(z + m_ref[...].T, 0.0)


def _sc_knn(yt_hbm, idx_hbm, out_hbm, idx_v, gbuf_a, gbuf_b, obuf_a, obuf_b,
            yt_sp, sem_a, sem_b):
    cid = lax.axis_index("c")
    sid = lax.axis_index("s")
    wid = sid * 2 + cid
    base = wid * NODES_PER_W
    pltpu.sync_copy(yt_hbm.at[pl.ds(sid * ROWS_PER_TILE, ROWS_PER_TILE)],
                    yt_sp.at[pl.ds(sid * ROWS_PER_TILE, ROWS_PER_TILE)])
    pltpu.sync_copy(idx_hbm.at[wid], idx_v)
    plsc.subcore_barrier()

    def issue(jc, buf, sem):
        pltpu.async_copy(yt_sp.at[idx_v.at[jc]], buf, sem)

    def wait(buf, sem):
        pltpu.make_async_copy(yt_sp.at[pl.ds(0, IDX_PER_CHUNK)], buf, sem).wait()

    def compute_store(j, buf, obuf):
        def node_body(i, _):
            r0 = i * K
            for g in range(D // 16):
                sl = pl.ds(g * 16, 16)
                m = buf[r0, sl]
                for k in range(1, K):
                    m = jnp.maximum(m, buf[r0 + k, sl])
                obuf[i, sl] = m
            return 0

        lax.fori_loop(0, CHUNK, node_body, 0)
        pltpu.sync_copy(obuf, out_hbm.at[pl.ds(base + j * CHUNK, CHUNK)])

    issue(0, gbuf_a, sem_a)
    issue(1, gbuf_b, sem_b)

    def chunk_body(jj, _):
        j0 = jj * 2
        wait(gbuf_a, sem_a)
        compute_store(j0, gbuf_a, obuf_a)
        issue(jnp.minimum(j0 + 2, NCHUNK - 1), gbuf_a, sem_a)
        wait(gbuf_b, sem_b)
        compute_store(j0 + 1, gbuf_b, obuf_b)
        issue(jnp.minimum(j0 + 3, NCHUNK - 1), gbuf_b, sem_b)
        return 0

    lax.fori_loop(0, NCHUNK // 2, chunk_body, 0)
    wait(gbuf_a, sem_a)
    wait(gbuf_b, sem_b)


def kernel(x, idx, W, b):
    x2 = x[0]
    idx_flat = idx[0].astype(jnp.int32).reshape(-1)
    idx3 = jnp.pad(idx_flat, (0, (NPAD - N) * K)).reshape(NW, NCHUNK, IDX_PER_CHUNK)
    b2 = b.reshape(1, D)

    yt = pl.pallas_call(
        _mm_body,
        grid=(NBLK,),
        in_specs=[
            pl.BlockSpec((D, NB), lambda i: (0, i)),
            pl.BlockSpec((D, 2 * D), lambda i: (0, 0)),
        ],
        out_specs=pl.BlockSpec((NB, D), lambda i: (i, 0)),
        out_shape=jax.ShapeDtypeStruct((NPAD, D), jnp.float32),
    )(x2, W)

    mesh = plsc.VectorSubcoreMesh(core_axis_name="c", subcore_axis_name="s")
    mt = pl.kernel(
        _sc_knn,
        out_type=jax.ShapeDtypeStruct((NPAD, D), jnp.float32),
        mesh=mesh,
        scratch_types=[
            pltpu.VMEM((NCHUNK, IDX_PER_CHUNK), jnp.int32),
            pltpu.VMEM((IDX_PER_CHUNK, D), jnp.float32),
            pltpu.VMEM((IDX_PER_CHUNK, D), jnp.float32),
            pltpu.VMEM((CHUNK, D), jnp.float32),
            pltpu.VMEM((CHUNK, D), jnp.float32),
            pltpu.VMEM_SHARED((NPAD, D), jnp.float32),
            pltpu.SemaphoreType.DMA,
            pltpu.SemaphoreType.DMA,
        ],
    )(yt, idx3)

    out = pl.pallas_call(
        _add_body,
        grid=(NBLK,),
        in_specs=[
            pl.BlockSpec((D, NB), lambda i: (0, i)),
            pl.BlockSpec((NB, D), lambda i: (i, 0)),
            pl.BlockSpec((D, 2 * D), lambda i: (0, 0)),
            pl.BlockSpec((1, D), lambda i: (0, 0)),
        ],
        out_specs=pl.BlockSpec((D, NB), lambda i: (0, i)),
        out_shape=jax.ShapeDtypeStruct((D, N), jnp.float32),
    )(x2, mt, W, b2)

    return out[None]
